# baseline (speedup 1.0000x reference)
import functools
import jax
import jax.numpy as jnp
from jax import lax
from jax.experimental import pallas as pl
from jax.experimental.pallas import tpu as pltpu, tpu_sc as plsc

_VOCAB = 1000000
_EMB = 32
_B = 16384
_L = 50
_WPB = _L * _EMB // 2
_WPR = _EMB // 2

_NC = 2
_NS = 16
_NW = _NC * _NS
_BPW = _B // _NW
_NB = 32
_NBATCH = _BPW // _NB


def _sc_kernel(vw_hbm, ctx_hbm, out_emb_hbm, pred_hbm,
               idxu, vv, uu, oo, semu):
    wid = lax.axis_index("s") * _NC + lax.axis_index("c")
    n_idx = _NB * _L

    def start_fetch(t, p):
        base = wid * _BPW + t * _NB
        pltpu.sync_copy(ctx_hbm.at[pl.ds(base * _L, n_idx)], idxu[p])
        pltpu.sync_copy(vw_hbm.at[pl.ds(base, _NB)], vv[p])
        pltpu.async_copy(out_emb_hbm.at[idxu[p]], uu[p], semu[p])

    def wait_fetch(p):
        pltpu.make_async_copy(out_emb_hbm.at[idxu[p]], uu[p], semu[p]).wait()

    def compute(t, p):
        u_v, v_v, o_v = uu[p], vv[p], oo[p]

        def b_body(b, carry2):
            iota = lax.broadcasted_iota(jnp.int32, (16,), 0)
            gbase = b * _WPB + iota
            bvec = jnp.full((16,), b, jnp.int32)
            acc = [jnp.zeros((16,), jnp.float32) for _ in range(4)]

            def word_chunk(g):
                x = plsc.load_gather(
                    u_v,
                    [lax.shift_right_logical(g, 4),
                     lax.bitwise_and(g, 15)])
                return plsc.unpack(
                    plsc.bitcast(x, jnp.bfloat16),
                    format=plsc.PackFormat.INTERLEAVED)

            for k in range(_WPR):
                wword = plsc.load_gather(
                    v_v, [bvec, jnp.full((16,), k, jnp.int32)])
                w_lo, w_hi = plsc.unpack(
                    plsc.bitcast(wword, jnp.bfloat16),
                    format=plsc.PackFormat.INTERLEAVED)
                for w, e in ((w_lo, 2 * k), (w_hi, 2 * k + 1)):
                    xa_ev, xa_od = word_chunk(gbase + e * 25)
                    xb_ev, xb_od = word_chunk(gbase + (e * 25 + 9))
                    acc[0] = acc[0] + w * xa_ev
                    acc[1] = acc[1] + w * xa_od
                    acc[2] = acc[2] + w * xb_ev
                    acc[3] = acc[3] + w * xb_od

            two = iota + iota
            base_l = b * _L + two
            plsc.store_scatter(o_v, [base_l], acc[0])
            plsc.store_scatter(o_v, [base_l + 1], acc[1])
            plsc.store_scatter(o_v, [base_l + 18], acc[2])
            plsc.store_scatter(o_v, [base_l + 19], acc[3])
            return carry2

        lax.fori_loop(0, _NB, b_body, 0)
        base = wid * _BPW + t * _NB
        pltpu.sync_copy(o_v, pred_hbm.at[pl.ds(base * _L, _NB * _L)])

    start_fetch(0, 0)

    def pair_body(k, carry):
        t0 = 2 * k
        start_fetch(t0 + 1, 1)
        wait_fetch(0)
        compute(t0, 0)
        @pl.when(k < _NBATCH // 2 - 1)
        def _():
            start_fetch(t0 + 2, 0)
        wait_fetch(1)
        compute(t0 + 1, 1)
        return carry

    lax.fori_loop(0, _NBATCH // 2, pair_body, 0)


@jax.jit
def _run(v_w, ctx_flat, out_w):
    mesh = plsc.VectorSubcoreMesh(core_axis_name="c", subcore_axis_name="s")
    f = pl.kernel(
        _sc_kernel,
        out_type=jax.ShapeDtypeStruct((_B * _L,), jnp.float32),
        mesh=mesh,
        scratch_types=[
            [pltpu.VMEM((_NB * _L,), jnp.int32) for _ in range(2)],
            [pltpu.VMEM((_NB, _WPR), jnp.int32) for _ in range(2)],
            [pltpu.VMEM((_NB * _L, _WPR), jnp.int32) for _ in range(2)],
            [pltpu.VMEM((_NB * _L,), jnp.float32) for _ in range(2)],
            [pltpu.SemaphoreType.DMA for _ in range(2)],
        ],
        compiler_params=pltpu.CompilerParams(
            use_tc_tiling_on_sc=False, needs_layout_passes=False),
    )
    return f(v_w, ctx_flat, out_w)


def _pack_words(table):
    bf = table.astype(jnp.bfloat16)
    lo = lax.bitcast_convert_type(bf[:, 0::2], jnp.uint16).astype(jnp.uint32)
    hi = lax.bitcast_convert_type(bf[:, 1::2], jnp.uint16).astype(jnp.uint32)
    return lax.bitcast_convert_type(lo | (hi << 16), jnp.int32)


def kernel(centers, contexts_negatives, in_emb, out_emb):
    centers_flat = centers.reshape(_B).astype(jnp.int32)
    ctx_flat = contexts_negatives.reshape(_B * _L).astype(jnp.int32)
    v_rows = jnp.take(in_emb, centers_flat, axis=0)
    v_w = _pack_words(v_rows)
    pred = _run(v_w, ctx_flat, _pack_words(out_emb))
    return pred.reshape(_B, 1, _L)


# speedup vs baseline: 2.7690x; 2.7690x over previous
import functools
import jax
import jax.numpy as jnp
from jax import lax
from jax.experimental import pallas as pl
from jax.experimental.pallas import tpu as pltpu, tpu_sc as plsc

_VOCAB = 1000000
_EMB = 32
_B = 16384
_L = 50

_NC = 2
_NS = 16
_NW = _NC * _NS
_BPW = _B // _NW
_NB = 32
_NBATCH = _BPW // _NB
_JOFF = (0, 16, 32, 34)


def _sc_kernel(v_hbm, ctx_hbm, out_emb_hbm, pred_hbm,
               idxu, vv, uu, oo, semu):
    wid = lax.axis_index("s") * _NC + lax.axis_index("c")
    n_idx = _NB * _L

    def start_fetch(t, p):
        base = wid * _BPW + t * _NB
        pltpu.sync_copy(ctx_hbm.at[pl.ds(base * _L, n_idx)], idxu[p])
        pltpu.sync_copy(v_hbm.at[pl.ds(base, _NB)], vv[p])
        pltpu.async_copy(out_emb_hbm.at[idxu[p]], uu[p], semu[p])

    def wait_fetch(p):
        pltpu.make_async_copy(out_emb_hbm.at[idxu[p]], uu[p], semu[p]).wait()

    def compute(t, p):
        u_v, v_v, o_v = uu[p], vv[p], oo[p]

        def b_body(b, carry2):
            fbase = b * (_L * _EMB) + lax.broadcasted_iota(
                jnp.int32, (16,), 0)
            bvec = jnp.full((16,), b, jnp.int32)
            accs = [jnp.zeros((16,), jnp.float32) for _ in range(4)]
            for e in range(_EMB):
                w = plsc.load_gather(
                    v_v, [bvec, jnp.full((16,), e, jnp.int32)])
                for j, joff in enumerate(_JOFF):
                    f = fbase + (e * _L + joff)
                    x = plsc.load_gather(
                        u_v,
                        [lax.shift_right_logical(f, 5),
                         lax.bitwise_and(f, 31)])
                    accs[j] = accs[j] + w * x
            oo_b = b * _L
            for j, joff in enumerate(_JOFF):
                o_v[pl.ds(oo_b + joff, 16)] = accs[j]
            return carry2

        lax.fori_loop(0, _NB, b_body, 0)
        base = wid * _BPW + t * _NB
        pltpu.sync_copy(o_v, pred_hbm.at[pl.ds(base * _L, _NB * _L)])

    start_fetch(0, 0)

    def pair_body(k, carry):
        t0 = 2 * k
        start_fetch(t0 + 1, 1)
        wait_fetch(0)
        compute(t0, 0)
        @pl.when(k < _NBATCH // 2 - 1)
        def _():
            start_fetch(t0 + 2, 0)
        wait_fetch(1)
        compute(t0 + 1, 1)
        return carry

    lax.fori_loop(0, _NBATCH // 2, pair_body, 0)


@jax.jit
def _run(v_rows, ctx_flat, out_emb):
    mesh = plsc.VectorSubcoreMesh(core_axis_name="c", subcore_axis_name="s")
    f = pl.kernel(
        _sc_kernel,
        out_type=jax.ShapeDtypeStruct((_B * _L,), jnp.float32),
        mesh=mesh,
        scratch_types=[
            [pltpu.VMEM((_NB * _L,), jnp.int32) for _ in range(2)],
            [pltpu.VMEM((_NB, _EMB), jnp.float32) for _ in range(2)],
            [pltpu.VMEM((_NB * _L, _EMB), jnp.float32) for _ in range(2)],
            [pltpu.VMEM((_NB * _L,), jnp.float32) for _ in range(2)],
            [pltpu.SemaphoreType.DMA for _ in range(2)],
        ],
        compiler_params=pltpu.CompilerParams(
            use_tc_tiling_on_sc=False, needs_layout_passes=False),
    )
    return f(v_rows, ctx_flat, out_emb)


def kernel(centers, contexts_negatives, in_emb, out_emb):
    centers_flat = centers.reshape(_B).astype(jnp.int32)
    ctx_flat = contexts_negatives.reshape(_B * _L).astype(jnp.int32)
    v_rows = jnp.take(in_emb, centers_flat, axis=0)
    pred = _run(v_rows, ctx_flat, out_emb)
    return pred.reshape(_B, 1, _L)
